# initial kernel scaffold (unmeasured)
import jax
import jax.numpy as jnp
from jax import lax
from jax.experimental import pallas as pl
from jax.experimental.pallas import tpu as pltpu

N_DEV = 32
N_TOK = 256
D = 128
H = 256
N_EXP = 64
CAP = 102.0


def kernel(x, router_W, route_idx, expert_W):
    del router_W

    def body(x_ref, idx_ref, ew_ref, out_ref, gw_ref, gr_ref,
             w_send_sems, w_recv_sems, r_send_sems, r_recv_sems):
        my = lax.axis_index("i")
        right = lax.rem(my + 1, N_DEV)
        left = lax.rem(my + N_DEV - 1, N_DEV)

        barrier = pltpu.get_barrier_semaphore()
        for nbr in (left, right):
            pl.semaphore_signal(
                barrier, inc=1,
                device_id=(nbr,), device_id_type=pl.DeviceIdType.MESH,
            )
        pl.semaphore_wait(barrier, 2)

        gw_ref[0, 0:D, :] = ew_ref[0]
        gw_ref[0, D:2 * D, :] = ew_ref[1]
        route_col = idx_ref[...].astype(jnp.float32)
        gr_ref[0] = route_col

        x_val = x_ref[...]

        exp_iota = lax.broadcasted_iota(jnp.float32, (N_TOK, N_EXP), 1)
        eqmat = (route_col == exp_iota).astype(jnp.float32)

        same = lax.dot_general(
            eqmat, eqmat, (((1,), (1,)), ((), ())),
            preferred_element_type=jnp.float32,
        )
        rows = lax.broadcasted_iota(jnp.float32, (N_TOK, N_TOK), 0)
        cols = lax.broadcasted_iota(jnp.float32, (N_TOK, N_TOK), 1)
        tril = (cols < rows).astype(jnp.float32)
        local_prefix = jnp.sum(same * tril, axis=1, keepdims=True)

        def chunk_contrib(k):
            origin = lax.rem(my - k + N_DEV, N_DEV)
            e0 = (2 * origin).astype(jnp.float32)
            w = gw_ref[k]
            sel0 = (route_col == e0).astype(jnp.float32)
            sel1 = (route_col == e0 + 1.0).astype(jnp.float32)
            c = jnp.dot(x_val * sel0, w[0:D, :],
                        preferred_element_type=jnp.float32)
            c = c + jnp.dot(x_val * sel1, w[D:2 * D, :],
                            preferred_element_type=jnp.float32)
            if k == 0:
                d_prior = jnp.zeros((N_TOK, 1), jnp.float32)
            else:
                mask = (my >= k).astype(jnp.float32)
                r_k = gr_ref[k]
                hist = jnp.sum((r_k == exp_iota).astype(jnp.float32),
                               axis=0, keepdims=True)
                d_prior = jnp.sum(eqmat * hist, axis=1, keepdims=True) * mask
            return c, d_prior

        acc = jnp.zeros((N_TOK, H), jnp.float32)
        prior = jnp.zeros((N_TOK, 1), jnp.float32)

        for h in range(N_DEV - 1):
            rdma_w = pltpu.make_async_remote_copy(
                src_ref=gw_ref.at[h], dst_ref=gw_ref.at[h + 1],
                send_sem=w_send_sems.at[h], recv_sem=w_recv_sems.at[h],
                device_id=(right,), device_id_type=pl.DeviceIdType.MESH,
            )
            rdma_r = pltpu.make_async_remote_copy(
                src_ref=gr_ref.at[h], dst_ref=gr_ref.at[h + 1],
                send_sem=r_send_sems.at[h], recv_sem=r_recv_sems.at[h],
                device_id=(right,), device_id_type=pl.DeviceIdType.MESH,
            )
            rdma_w.start()
            rdma_r.start()
            c, dp = chunk_contrib(h)
            acc = acc + c
            prior = prior + dp
            rdma_w.wait_recv()
            rdma_r.wait_recv()
            rdma_w.wait_send()
            rdma_r.wait_send()

        c, dp = chunk_contrib(N_DEV - 1)
        acc = acc + c
        prior = prior + dp

        keep = ((prior + local_prefix) < CAP).astype(jnp.float32)
        out_ref[...] = acc * keep

    return pl.pallas_call(
        body,
        out_shape=jax.ShapeDtypeStruct((N_TOK, H), jnp.float32),
        in_specs=[
            pl.BlockSpec(memory_space=pltpu.VMEM),
            pl.BlockSpec(memory_space=pltpu.VMEM),
            pl.BlockSpec(memory_space=pltpu.VMEM),
        ],
        out_specs=pl.BlockSpec(memory_space=pltpu.VMEM),
        scratch_shapes=[
            pltpu.VMEM((N_DEV, 2 * D, H), jnp.float32),
            pltpu.VMEM((N_DEV, N_TOK, 1), jnp.float32),
            pltpu.SemaphoreType.DMA((N_DEV - 1,)),
            pltpu.SemaphoreType.DMA((N_DEV - 1,)),
            pltpu.SemaphoreType.DMA((N_DEV - 1,)),
            pltpu.SemaphoreType.DMA((N_DEV - 1,)),
        ],
        compiler_params=pltpu.CompilerParams(collective_id=0),
    )(x, route_idx, expert_W)


# baseline (device time: 195612 ns/iter reference)
import jax
import jax.numpy as jnp
from jax import lax
from jax.experimental import pallas as pl
from jax.experimental.pallas import tpu as pltpu

N_DEV = 32
N_TOK = 256
D = 128
H = 256
N_EXP = 64
CAP = 102.0


def kernel(x, router_W, route_idx, expert_W):
    del router_W

    def body(x_ref, idx_ref, ew_ref, out_ref, gw_ref, gr_ref,
             w_send_sems, w_recv_sems, r_send_sems, r_recv_sems):
        my = lax.axis_index("i")
        right = lax.rem(my + 1, N_DEV)
        left = lax.rem(my + N_DEV - 1, N_DEV)

        barrier = pltpu.get_barrier_semaphore()
        for nbr in (left, right):
            pl.semaphore_signal(
                barrier, inc=1,
                device_id=(nbr,), device_id_type=pl.DeviceIdType.MESH,
            )
        pl.semaphore_wait(barrier, 2)

        gw_ref[0, 0:D, :] = ew_ref[0]
        gw_ref[0, D:2 * D, :] = ew_ref[1]
        route_i32 = idx_ref[...]
        gr_ref[0] = route_i32.astype(jnp.float32)

        x_val = x_ref[...]

        exp_iota = lax.broadcasted_iota(jnp.int32, (N_TOK, N_EXP), 1)
        eqmat = (route_i32 == exp_iota).astype(jnp.float32)

        same = lax.dot_general(
            eqmat, eqmat, (((1,), (1,)), ((), ())),
            preferred_element_type=jnp.float32,
        )
        rows = lax.broadcasted_iota(jnp.int32, (N_TOK, N_TOK), 0)
        cols = lax.broadcasted_iota(jnp.int32, (N_TOK, N_TOK), 1)
        tril = (cols < rows).astype(jnp.float32)
        local_prefix = jnp.sum(same * tril, axis=1, keepdims=True)

        def chunk_contrib(k):
            origin = lax.rem(my - k + N_DEV, N_DEV)
            e0 = (2 * origin).astype(jnp.int32)
            w = gw_ref[k]
            sel0 = (route_i32 == e0).astype(jnp.float32)
            sel1 = (route_i32 == e0 + 1).astype(jnp.float32)
            c = jnp.dot(x_val * sel0, w[0:D, :],
                        preferred_element_type=jnp.float32)
            c = c + jnp.dot(x_val * sel1, w[D:2 * D, :],
                            preferred_element_type=jnp.float32)
            if k == 0:
                d_prior = jnp.zeros((N_TOK, 1), jnp.float32)
            else:
                mask = (my >= k).astype(jnp.float32)
                r_k = gr_ref[k].astype(jnp.int32)
                hist = jnp.sum((r_k == exp_iota).astype(jnp.float32),
                               axis=0, keepdims=True)
                d_prior = jnp.sum(eqmat * hist, axis=1, keepdims=True) * mask
            return c, d_prior

        acc = jnp.zeros((N_TOK, H), jnp.float32)
        prior = jnp.zeros((N_TOK, 1), jnp.float32)

        for h in range(N_DEV - 1):
            rdma_w = pltpu.make_async_remote_copy(
                src_ref=gw_ref.at[h], dst_ref=gw_ref.at[h + 1],
                send_sem=w_send_sems.at[h], recv_sem=w_recv_sems.at[h],
                device_id=(right,), device_id_type=pl.DeviceIdType.MESH,
            )
            rdma_r = pltpu.make_async_remote_copy(
                src_ref=gr_ref.at[h], dst_ref=gr_ref.at[h + 1],
                send_sem=r_send_sems.at[h], recv_sem=r_recv_sems.at[h],
                device_id=(right,), device_id_type=pl.DeviceIdType.MESH,
            )
            rdma_w.start()
            rdma_r.start()
            c, dp = chunk_contrib(h)
            acc = acc + c
            prior = prior + dp
            rdma_w.wait_recv()
            rdma_r.wait_recv()
            rdma_w.wait_send()
            rdma_r.wait_send()

        c, dp = chunk_contrib(N_DEV - 1)
        acc = acc + c
        prior = prior + dp

        keep = ((prior + local_prefix) < CAP).astype(jnp.float32)
        out_ref[...] = acc * keep

    return pl.pallas_call(
        body,
        out_shape=jax.ShapeDtypeStruct((N_TOK, H), jnp.float32),
        in_specs=[
            pl.BlockSpec(memory_space=pltpu.VMEM),
            pl.BlockSpec(memory_space=pltpu.VMEM),
            pl.BlockSpec(memory_space=pltpu.VMEM),
        ],
        out_specs=pl.BlockSpec(memory_space=pltpu.VMEM),
        scratch_shapes=[
            pltpu.VMEM((N_DEV, 2 * D, H), jnp.float32),
            pltpu.VMEM((N_DEV, N_TOK, 1), jnp.float32),
            pltpu.SemaphoreType.DMA((N_DEV - 1,)),
            pltpu.SemaphoreType.DMA((N_DEV - 1,)),
            pltpu.SemaphoreType.DMA((N_DEV - 1,)),
            pltpu.SemaphoreType.DMA((N_DEV - 1,)),
        ],
        compiler_params=pltpu.CompilerParams(collective_id=0),
    )(x, route_idx, expert_W)


# device time: 122645 ns/iter; 1.5949x vs baseline; 1.5949x over previous
import jax
import jax.numpy as jnp
from jax import lax
from jax.experimental import pallas as pl
from jax.experimental.pallas import tpu as pltpu

N_DEV = 32
N_TOK = 256
D = 128
H = 256
N_EXP = 64
CAP = 102.0

R_HOPS = 16
L_HOPS = 15
SLOT_ROWS = 2 * D + 8


def kernel(x, router_W, route_idx, expert_W):
    del router_W

    def body(x_ref, idx_ref, ew_ref, out_ref, gw_ref,
             r_send_sems, r_recv_sems, l_send_sems, l_recv_sems):
        my = lax.axis_index("i")
        right = lax.rem(my + 1, N_DEV)
        left = lax.rem(my + N_DEV - 1, N_DEV)

        barrier = pltpu.get_barrier_semaphore()
        for nbr in (left, right):
            pl.semaphore_signal(
                barrier, inc=1,
                device_id=(nbr,), device_id_type=pl.DeviceIdType.MESH,
            )
        pl.semaphore_wait(barrier, 2)

        x_val = x_ref[...]
        route_i32 = idx_ref[...]

        exp_iota = lax.broadcasted_iota(jnp.int32, (N_TOK, N_EXP), 1)
        eqmat = (route_i32 == exp_iota).astype(jnp.float32)

        gw_ref[0, 0:D, :] = ew_ref[0]
        gw_ref[0, D:2 * D, :] = ew_ref[1]
        gw_ref[0, 2 * D:SLOT_ROWS, :] = jnp.zeros(
            (SLOT_ROWS - 2 * D, H), jnp.float32)
        gw_ref[0, 2 * D:2 * D + 1, 0:N_EXP] = jnp.sum(
            eqmat, axis=0, keepdims=True)

        same = lax.dot_general(
            eqmat, eqmat, (((1,), (1,)), ((), ())),
            preferred_element_type=jnp.float32,
        )
        rows = lax.broadcasted_iota(jnp.int32, (N_TOK, N_TOK), 0)
        cols = lax.broadcasted_iota(jnp.int32, (N_TOK, N_TOK), 1)
        tril = (cols < rows).astype(jnp.float32)
        local_prefix = jnp.sum(same * tril, axis=1, keepdims=True)

        def chunk_contrib(j):
            origin = lax.rem(my - j + N_DEV, N_DEV)
            e0 = (2 * origin).astype(jnp.int32)
            w = gw_ref[j]
            sel0 = (route_i32 == e0).astype(jnp.float32)
            sel1 = (route_i32 == e0 + 1).astype(jnp.float32)
            c = jnp.dot(x_val * sel0, w[0:D, :],
                        preferred_element_type=jnp.float32)
            c = c + jnp.dot(x_val * sel1, w[D:2 * D, :],
                            preferred_element_type=jnp.float32)
            if j == 0:
                d_prior = jnp.zeros((N_TOK, 1), jnp.float32)
            else:
                mask = (my >= j).astype(jnp.float32)
                hist = w[2 * D:2 * D + 1, 0:N_EXP]
                d_prior = jnp.sum(eqmat * hist, axis=1, keepdims=True) * mask
            return c, d_prior

        acc = jnp.zeros((N_TOK, H), jnp.float32)
        prior = jnp.zeros((N_TOK, 1), jnp.float32)

        for h in range(R_HOPS):
            rdma_r = pltpu.make_async_remote_copy(
                src_ref=gw_ref.at[h], dst_ref=gw_ref.at[h + 1],
                send_sem=r_send_sems.at[h], recv_sem=r_recv_sems.at[h],
                device_id=(right,), device_id_type=pl.DeviceIdType.MESH,
            )
            rdma_r.start()
            if h < L_HOPS:
                rdma_l = pltpu.make_async_remote_copy(
                    src_ref=gw_ref.at[0 if h == 0 else N_DEV - h],
                    dst_ref=gw_ref.at[N_DEV - 1 - h],
                    send_sem=l_send_sems.at[h], recv_sem=l_recv_sems.at[h],
                    device_id=(left,), device_id_type=pl.DeviceIdType.MESH,
                )
                rdma_l.start()
            c, dp = chunk_contrib(h)
            acc = acc + c
            prior = prior + dp
            if h >= 1:
                c, dp = chunk_contrib(N_DEV - h)
                acc = acc + c
                prior = prior + dp
            rdma_r.wait_recv()
            rdma_r.wait_send()
            if h < L_HOPS:
                rdma_l.wait_recv()
                rdma_l.wait_send()

        c, dp = chunk_contrib(R_HOPS)
        acc = acc + c
        prior = prior + dp

        keep = ((prior + local_prefix) < CAP).astype(jnp.float32)
        out_ref[...] = acc * keep

    return pl.pallas_call(
        body,
        out_shape=jax.ShapeDtypeStruct((N_TOK, H), jnp.float32),
        in_specs=[
            pl.BlockSpec(memory_space=pltpu.VMEM),
            pl.BlockSpec(memory_space=pltpu.VMEM),
            pl.BlockSpec(memory_space=pltpu.VMEM),
        ],
        out_specs=pl.BlockSpec(memory_space=pltpu.VMEM),
        scratch_shapes=[
            pltpu.VMEM((N_DEV, SLOT_ROWS, H), jnp.float32),
            pltpu.SemaphoreType.DMA((R_HOPS,)),
            pltpu.SemaphoreType.DMA((R_HOPS,)),
            pltpu.SemaphoreType.DMA((L_HOPS,)),
            pltpu.SemaphoreType.DMA((L_HOPS,)),
        ],
        compiler_params=pltpu.CompilerParams(collective_id=0),
    )(x, route_idx, expert_W)
